# baseline (speedup 1.0000x reference)
import jax
import jax.numpy as jnp
from jax.experimental import pallas as pl
from jax.experimental.pallas import tpu as pltpu

_ALPHA = 0.25
_GAMMA = 2.0


def _image_block(inst, c_raw, reg_n, a, bbox, bt):
    nblk = a.shape[1]
    m = bbox.shape[0]

    a0 = a[0:1]
    a1 = a[1:2]
    a2 = a[2:3]
    a3 = a[3:4]
    area_a = (a2 - a0) * (a3 - a1)

    b0 = bbox[:, 0:1]
    b1 = bbox[:, 1:2]
    b2 = bbox[:, 2:3]
    b3 = bbox[:, 3:4]
    area_b = (b2 - b0) * (b3 - b1)

    iw = jnp.minimum(a3, b2) - jnp.maximum(a1, b0)
    ih = jnp.minimum(a2, b3) - jnp.maximum(a0, b1)
    iw = jnp.maximum(iw, 0.0)
    ih = jnp.maximum(ih, 0.0)
    inter = iw * ih
    ua = jnp.maximum(area_a + area_b - inter, 1e-8)
    iou = inter / ua

    iou_max = jnp.max(iou, axis=0, keepdims=True)
    box_idx = jax.lax.broadcasted_iota(jnp.int32, (m, nblk), 0)
    amax = jnp.min(jnp.where(iou == iou_max, box_idx, m), axis=0, keepdims=True)
    onehot_box = (box_idx == amax).astype(jnp.float32)
    assigned = jnp.dot(bt, onehot_box, preferred_element_type=jnp.float32)

    pos = iou_max >= 0.5
    posi = iou_max >= 0.3
    posf = pos.astype(jnp.float32)
    np_p = jnp.sum(posf)
    npi_p = jnp.sum(posi.astype(jnp.float32))

    inst = jnp.clip(inst, 0.0001, 1.0 - 0.0001)
    flag = assigned[5:6]
    ti = jnp.where(posi & (flag == 1.0), 1.0,
                   jnp.where(posi & (flag == 0.0), 0.0, -1.0))
    q = jnp.where(ti == 1.0, 1.0 - inst, inst)
    il_p = jnp.sum(jnp.where(ti != -1.0,
                             (0.5 * (q * q)) * (-jnp.log(1.0 - q)), 0.0))

    aw = a3 - a1
    ah = a2 - a0
    acx = a1 + 0.5 * aw
    acy = a0 + 0.5 * ah
    g0 = assigned[0:1]
    g1 = assigned[1:2]
    g2 = assigned[2:3]
    g3 = assigned[3:4]
    gw = jnp.maximum(g2 - g0, 1.0)
    gh = jnp.maximum(g3 - g1, 1.0)
    gcx = g0 + 0.5 * (g2 - g0)
    gcy = g1 + 0.5 * (g3 - g1)
    tdx = (gcx - acx) / aw
    tdy = (gcy - acy) / ah
    tdw = jnp.log(gw / aw)
    tdh = jnp.log(gh / ah)
    reg = jnp.transpose(reg_n)
    rl_p = jnp.float32(0.0)
    for k, t_row in enumerate((tdy, tdx, tdh, tdw)):
        d = jnp.abs(t_row - reg[k:k + 1])
        l = jnp.where(d <= 1.0 / 9.0, 4.5 * d * d, d - 0.5 / 9.0)
        rl_p += jnp.sum(l * posf)

    c = jnp.clip(c_raw, 0.0001, 1.0 - 0.0001)
    ncls = c.shape[1]
    cls_id = assigned[4:5].astype(jnp.int32)
    lt04f = (iou_max < 0.4).astype(jnp.float32)
    activef = jnp.maximum(posf, lt04f)
    l0 = (0.75 * (c * c)) * (-jnp.log(1.0 - c))
    cls_id_s = cls_id.reshape(nblk, 1)
    lane = jax.lax.broadcasted_iota(jnp.int32, (nblk, ncls), 1)
    csel = jnp.where(lane == cls_id_s, c, 0.0)
    ones_col = jnp.ones((ncls, 1), dtype=jnp.float32)
    rowsum0 = jnp.dot(l0, ones_col, preferred_element_type=jnp.float32)
    ck_s = jnp.dot(csel, ones_col, preferred_element_type=jnp.float32)
    cl_main = jnp.dot(activef, rowsum0,
                      preferred_element_type=jnp.float32)[0, 0]
    ck = jnp.maximum(ck_s.reshape(1, nblk), 0.0001)
    corrval = (0.25 * ((1.0 - ck) * (1.0 - ck))) * (-jnp.log(ck)) \
        - (0.75 * (ck * ck)) * (-jnp.log(1.0 - ck))
    cl_p = cl_main + jnp.sum(posf * corrval)

    return il_p, cl_p, rl_p, np_p, npi_p


def _make_kernel(bi):
    def _fl_kernel(inst_ref, cls_ref, reg_ref, anc_ref, ann_ref, annT_ref,
                   il_ref, cl_ref, rl_ref, np_ref, npi_ref):
        nb = pl.program_id(1)
        a = anc_ref[0, 0]
        parts = []
        for i in range(bi):
            parts.append(_image_block(inst_ref[i, 0], cls_ref[i, 0],
                                      reg_ref[i, 0], a,
                                      ann_ref[i], annT_ref[i]))
        outs = (il_ref, cl_ref, rl_ref, np_ref, npi_ref)
        stacked = [jnp.stack([p[k] for p in parts]).reshape(bi, 1, 1)
                   for k in range(5)]

        @pl.when(nb == 0)
        def _():
            for ref, v in zip(outs, stacked):
                ref[...] = v

        @pl.when(nb != 0)
        def _():
            for ref, v in zip(outs, stacked):
                ref[...] += v

    return _fl_kernel


def _run(instances, classifications, regressions, anchors, annotations,
         nblk=2500, bi=2, interpret=False):
    B, N, C = classifications.shape
    M = annotations.shape[1]
    NB = N // nblk
    grid = (B // bi, NB)
    inst4 = instances.reshape(B, NB, 1, nblk)
    cls4 = classifications.reshape(B, NB, nblk, C)
    reg4 = regressions.reshape(B, NB, nblk, 4)
    anc4 = anchors.reshape(1, NB, nblk, 4).transpose(0, 1, 3, 2)
    annT = annotations.transpose(0, 2, 1)
    outs = pl.pallas_call(
        _make_kernel(bi),
        grid=grid,
        in_specs=[
            pl.BlockSpec((bi, 1, 1, nblk), lambda j, b: (j, b, 0, 0)),
            pl.BlockSpec((bi, 1, nblk, C), lambda j, b: (j, b, 0, 0)),
            pl.BlockSpec((bi, 1, nblk, 4), lambda j, b: (j, b, 0, 0)),
            pl.BlockSpec((1, 1, 4, nblk), lambda j, b: (0, b, 0, 0)),
            pl.BlockSpec((bi, M, 6), lambda j, b: (j, 0, 0)),
            pl.BlockSpec((bi, 6, M), lambda j, b: (j, 0, 0)),
        ],
        out_specs=[pl.BlockSpec((bi, 1, 1), lambda j, b: (j, 0, 0))] * 5,
        out_shape=[jax.ShapeDtypeStruct((B, 1, 1), jnp.float32)] * 5,
        compiler_params=pltpu.CompilerParams(
            dimension_semantics=("parallel", "arbitrary")),
        interpret=interpret,
    )(inst4, cls4, reg4, anc4, annotations, annT)
    il_s, cl_s, rl_s, npos, nposi = [o[:, 0, 0] for o in outs]
    il = (il_s / jnp.maximum(nposi, 1.0)).mean(keepdims=True)
    cl = (cl_s / jnp.maximum(npos, 1.0)).mean(keepdims=True)
    rl = (rl_s / jnp.maximum(npos * 4.0, 1.0)).mean(keepdims=True)
    return (il, cl, rl)


def kernel(instances, classifications, regressions, anchors, annotations):
    return _run(instances, classifications, regressions, anchors, annotations)


# speedup vs baseline: 1.1347x; 1.1347x over previous
import jax
import jax.numpy as jnp
from jax.experimental import pallas as pl
from jax.experimental.pallas import tpu as pltpu

_ALPHA = 0.25
_GAMMA = 2.0


def _image_block(inst, c_raw, reg, a, bbox, bt):
    nblk = a.shape[1]
    m = bbox.shape[0]

    a0 = a[0:1]
    a1 = a[1:2]
    a2 = a[2:3]
    a3 = a[3:4]
    area_a = (a2 - a0) * (a3 - a1)

    b0 = bbox[:, 0:1]
    b1 = bbox[:, 1:2]
    b2 = bbox[:, 2:3]
    b3 = bbox[:, 3:4]
    area_b = (b2 - b0) * (b3 - b1)

    iw = jnp.minimum(a3, b2) - jnp.maximum(a1, b0)
    ih = jnp.minimum(a2, b3) - jnp.maximum(a0, b1)
    iw = jnp.maximum(iw, 0.0)
    ih = jnp.maximum(ih, 0.0)
    inter = iw * ih
    ua = jnp.maximum(area_a + area_b - inter, 1e-8)
    iou = inter / ua

    iou_max = jnp.max(iou, axis=0, keepdims=True)
    box_idx = jax.lax.broadcasted_iota(jnp.int32, (m, nblk), 0)
    amax = jnp.min(jnp.where(iou == iou_max, box_idx, m), axis=0, keepdims=True)
    onehot_box = (box_idx == amax).astype(jnp.float32)
    assigned = jnp.dot(bt, onehot_box, preferred_element_type=jnp.float32)

    pos = iou_max >= 0.5
    posi = iou_max >= 0.3
    posf = pos.astype(jnp.float32)
    np_p = jnp.sum(posf)
    npi_p = jnp.sum(posi.astype(jnp.float32))

    inst = jnp.clip(inst, 0.0001, 1.0 - 0.0001)
    flag = assigned[5:6]
    ti = jnp.where(posi & (flag == 1.0), 1.0,
                   jnp.where(posi & (flag == 0.0), 0.0, -1.0))
    q = jnp.where(ti == 1.0, 1.0 - inst, inst)
    il_p = jnp.sum(jnp.where(ti != -1.0,
                             (0.5 * (q * q)) * (-jnp.log(1.0 - q)), 0.0))

    aw = a3 - a1
    ah = a2 - a0
    acx = a1 + 0.5 * aw
    acy = a0 + 0.5 * ah
    g0 = assigned[0:1]
    g1 = assigned[1:2]
    g2 = assigned[2:3]
    g3 = assigned[3:4]
    gw = jnp.maximum(g2 - g0, 1.0)
    gh = jnp.maximum(g3 - g1, 1.0)
    gcx = g0 + 0.5 * (g2 - g0)
    gcy = g1 + 0.5 * (g3 - g1)
    tdx = (gcx - acx) / aw
    tdy = (gcy - acy) / ah
    tdw = jnp.log(gw / aw)
    tdh = jnp.log(gh / ah)
    rl_p = jnp.float32(0.0)
    for k, t_row in enumerate((tdy, tdx, tdh, tdw)):
        d = jnp.abs(t_row - reg[k:k + 1])
        l = jnp.where(d <= 1.0 / 9.0, 4.5 * d * d, d - 0.5 / 9.0)
        rl_p += jnp.sum(l * posf)

    c = jnp.clip(c_raw, 0.0001, 1.0 - 0.0001)
    ncls = c.shape[1]
    cls_id = assigned[4:5].astype(jnp.int32)
    lt04f = (iou_max < 0.4).astype(jnp.float32)
    activef = jnp.maximum(posf, lt04f)
    l0 = (0.75 * (c * c)) * (-jnp.log(1.0 - c))
    cls_id_s = cls_id.reshape(nblk, 1)
    lane = jax.lax.broadcasted_iota(jnp.int32, (nblk, ncls), 1)
    csel = jnp.where(lane == cls_id_s, c, 0.0)
    ones_col = jnp.ones((ncls, 1), dtype=jnp.float32)
    rowsum0 = jnp.dot(l0, ones_col, preferred_element_type=jnp.float32)
    ck_s = jnp.dot(csel, ones_col, preferred_element_type=jnp.float32)
    cl_main = jnp.dot(activef, rowsum0,
                      preferred_element_type=jnp.float32)[0, 0]
    ck = jnp.maximum(ck_s.reshape(1, nblk), 0.0001)
    corrval = (0.25 * ((1.0 - ck) * (1.0 - ck))) * (-jnp.log(ck)) \
        - (0.75 * (ck * ck)) * (-jnp.log(1.0 - ck))
    cl_p = cl_main + jnp.sum(posf * corrval)

    return il_p, cl_p, rl_p, np_p, npi_p


def _make_kernel(bi):
    def _fl_kernel(inst_ref, cls_ref, reg_ref, anc_ref, ann_ref, annT_ref,
                   il_ref, cl_ref, rl_ref, np_ref, npi_ref):
        nb = pl.program_id(1)
        a = anc_ref[0, 0]
        parts = []
        for i in range(bi):
            parts.append(_image_block(inst_ref[i, 0], cls_ref[i, 0],
                                      reg_ref[i, 0], a,
                                      ann_ref[i], annT_ref[i]))
        outs = (il_ref, cl_ref, rl_ref, np_ref, npi_ref)
        stacked = [jnp.stack([p[k] for p in parts]).reshape(bi, 1, 1)
                   for k in range(5)]

        @pl.when(nb == 0)
        def _():
            for ref, v in zip(outs, stacked):
                ref[...] = v

        @pl.when(nb != 0)
        def _():
            for ref, v in zip(outs, stacked):
                ref[...] += v

    return _fl_kernel


def _run(instances, classifications, regressions, anchors, annotations,
         nblk=2500, bi=4, interpret=False):
    B, N, C = classifications.shape
    M = annotations.shape[1]
    NB = N // nblk
    grid = (B // bi, NB)
    inst4 = instances.reshape(B, NB, 1, nblk)
    cls4 = classifications.reshape(B, NB, nblk, C)
    reg4 = regressions.reshape(B, NB, nblk, 4).transpose(0, 1, 3, 2)
    anc4 = anchors.reshape(1, NB, nblk, 4).transpose(0, 1, 3, 2)
    annT = annotations.transpose(0, 2, 1)
    outs = pl.pallas_call(
        _make_kernel(bi),
        grid=grid,
        in_specs=[
            pl.BlockSpec((bi, 1, 1, nblk), lambda j, b: (j, b, 0, 0)),
            pl.BlockSpec((bi, 1, nblk, C), lambda j, b: (j, b, 0, 0)),
            pl.BlockSpec((bi, 1, 4, nblk), lambda j, b: (j, b, 0, 0)),
            pl.BlockSpec((1, 1, 4, nblk), lambda j, b: (0, b, 0, 0)),
            pl.BlockSpec((bi, M, 6), lambda j, b: (j, 0, 0)),
            pl.BlockSpec((bi, 6, M), lambda j, b: (j, 0, 0)),
        ],
        out_specs=[pl.BlockSpec((bi, 1, 1), lambda j, b: (j, 0, 0))] * 5,
        out_shape=[jax.ShapeDtypeStruct((B, 1, 1), jnp.float32)] * 5,
        compiler_params=pltpu.CompilerParams(
            dimension_semantics=("parallel", "arbitrary")),
        interpret=interpret,
    )(inst4, cls4, reg4, anc4, annotations, annT)
    il_s, cl_s, rl_s, npos, nposi = [o[:, 0, 0] for o in outs]
    il = (il_s / jnp.maximum(nposi, 1.0)).mean(keepdims=True)
    cl = (cl_s / jnp.maximum(npos, 1.0)).mean(keepdims=True)
    rl = (rl_s / jnp.maximum(npos * 4.0, 1.0)).mean(keepdims=True)
    return (il, cl, rl)


def kernel(instances, classifications, regressions, anchors, annotations):
    return _run(instances, classifications, regressions, anchors, annotations)


# speedup vs baseline: 1.1421x; 1.0065x over previous
import jax
import jax.numpy as jnp
from jax.experimental import pallas as pl
from jax.experimental.pallas import tpu as pltpu

_ALPHA = 0.25
_GAMMA = 2.0


def _image_block(inst, c_raw, reg, a, bbox, bt):
    nblk = a.shape[1]
    m = bbox.shape[0]

    a0 = a[0:1]
    a1 = a[1:2]
    a2 = a[2:3]
    a3 = a[3:4]
    area_a = (a2 - a0) * (a3 - a1)

    b0 = bbox[:, 0:1]
    b1 = bbox[:, 1:2]
    b2 = bbox[:, 2:3]
    b3 = bbox[:, 3:4]
    area_b = (b2 - b0) * (b3 - b1)

    iw = jnp.minimum(a3, b2) - jnp.maximum(a1, b0)
    ih = jnp.minimum(a2, b3) - jnp.maximum(a0, b1)
    iw = jnp.maximum(iw, 0.0)
    ih = jnp.maximum(ih, 0.0)
    inter = iw * ih
    ua = jnp.maximum(area_a + area_b - inter, 1e-8)
    iou = inter / ua

    iou_max = jnp.max(iou, axis=0, keepdims=True)
    box_idx = jax.lax.broadcasted_iota(jnp.int32, (m, nblk), 0)
    amax = jnp.min(jnp.where(iou == iou_max, box_idx, m), axis=0, keepdims=True)
    onehot_box = (box_idx == amax).astype(jnp.float32)
    assigned = jnp.dot(bt, onehot_box, preferred_element_type=jnp.float32)

    pos = iou_max >= 0.5
    posi = iou_max >= 0.3
    posf = pos.astype(jnp.float32)
    np_p = jnp.sum(posf)
    npi_p = jnp.sum(posi.astype(jnp.float32))

    inst = jnp.clip(inst, 0.0001, 1.0 - 0.0001)
    flag = assigned[5:6]
    ti = jnp.where(posi & (flag == 1.0), 1.0,
                   jnp.where(posi & (flag == 0.0), 0.0, -1.0))
    q = jnp.where(ti == 1.0, 1.0 - inst, inst)
    il_p = jnp.sum(jnp.where(ti != -1.0,
                             (0.5 * (q * q)) * (-jnp.log(1.0 - q)), 0.0))

    aw = a3 - a1
    ah = a2 - a0
    acx = a1 + 0.5 * aw
    acy = a0 + 0.5 * ah
    g0 = assigned[0:1]
    g1 = assigned[1:2]
    g2 = assigned[2:3]
    g3 = assigned[3:4]
    gw = jnp.maximum(g2 - g0, 1.0)
    gh = jnp.maximum(g3 - g1, 1.0)
    gcx = g0 + 0.5 * (g2 - g0)
    gcy = g1 + 0.5 * (g3 - g1)
    tdx = (gcx - acx) / aw
    tdy = (gcy - acy) / ah
    tdw = jnp.log(gw / aw)
    tdh = jnp.log(gh / ah)
    rl_p = jnp.float32(0.0)
    for k, t_row in enumerate((tdy, tdx, tdh, tdw)):
        d = jnp.abs(t_row - reg[k:k + 1])
        l = jnp.where(d <= 1.0 / 9.0, 4.5 * d * d, d - 0.5 / 9.0)
        rl_p += jnp.sum(l * posf)

    c = jnp.clip(c_raw, 0.0001, 1.0 - 0.0001)
    ncls = c.shape[1]
    cls_id = assigned[4:5].astype(jnp.int32)
    lt04f = (iou_max < 0.4).astype(jnp.float32)
    activef = jnp.maximum(posf, lt04f)
    l0 = (0.75 * (c * c)) * (-jnp.log(1.0 - c))
    cls_id_s = cls_id.reshape(nblk, 1)
    lane = jax.lax.broadcasted_iota(jnp.int32, (nblk, ncls), 1)
    csel = jnp.where(lane == cls_id_s, c, 0.0)
    ones_col = jnp.ones((ncls, 1), dtype=jnp.float32)
    rowsum0 = jnp.dot(l0, ones_col, preferred_element_type=jnp.float32)
    ck_s = jnp.dot(csel, ones_col, preferred_element_type=jnp.float32)
    cl_main = jnp.dot(activef, rowsum0,
                      preferred_element_type=jnp.float32)[0, 0]
    ck = jnp.maximum(ck_s.reshape(1, nblk), 0.0001)
    corrval = (0.25 * ((1.0 - ck) * (1.0 - ck))) * (-jnp.log(ck)) \
        - (0.75 * (ck * ck)) * (-jnp.log(1.0 - ck))
    cl_p = cl_main + jnp.sum(posf * corrval)

    return il_p, cl_p, rl_p, np_p, npi_p


def _make_kernel(bi):
    def _fl_kernel(inst_ref, cls_ref, reg_ref, anc_ref, ann_ref, annT_ref,
                   il_ref, cl_ref, rl_ref, np_ref, npi_ref):
        nb = pl.program_id(1)
        a = anc_ref[0, 0]
        parts = []
        for i in range(bi):
            parts.append(_image_block(inst_ref[i, 0], cls_ref[i, 0],
                                      reg_ref[i, 0], a,
                                      ann_ref[i], annT_ref[i]))
        outs = (il_ref, cl_ref, rl_ref, np_ref, npi_ref)
        stacked = [jnp.stack([p[k] for p in parts]).reshape(bi, 1, 1)
                   for k in range(5)]

        @pl.when(nb == 0)
        def _():
            for ref, v in zip(outs, stacked):
                ref[...] = v

        @pl.when(nb != 0)
        def _():
            for ref, v in zip(outs, stacked):
                ref[...] += v

    return _fl_kernel


def _run(instances, classifications, regressions, anchors, annotations,
         nblk=2500, bi=8, interpret=False):
    B, N, C = classifications.shape
    M = annotations.shape[1]
    NB = N // nblk
    grid = (B // bi, NB)
    inst4 = instances.reshape(B, NB, 1, nblk)
    cls4 = classifications.reshape(B, NB, nblk, C)
    reg4 = regressions.reshape(B, NB, nblk, 4).transpose(0, 1, 3, 2)
    anc4 = anchors.reshape(1, NB, nblk, 4).transpose(0, 1, 3, 2)
    annT = annotations.transpose(0, 2, 1)
    outs = pl.pallas_call(
        _make_kernel(bi),
        grid=grid,
        in_specs=[
            pl.BlockSpec((bi, 1, 1, nblk), lambda j, b: (j, b, 0, 0)),
            pl.BlockSpec((bi, 1, nblk, C), lambda j, b: (j, b, 0, 0)),
            pl.BlockSpec((bi, 1, 4, nblk), lambda j, b: (j, b, 0, 0)),
            pl.BlockSpec((1, 1, 4, nblk), lambda j, b: (0, b, 0, 0)),
            pl.BlockSpec((bi, M, 6), lambda j, b: (j, 0, 0)),
            pl.BlockSpec((bi, 6, M), lambda j, b: (j, 0, 0)),
        ],
        out_specs=[pl.BlockSpec((bi, 1, 1), lambda j, b: (j, 0, 0))] * 5,
        out_shape=[jax.ShapeDtypeStruct((B, 1, 1), jnp.float32)] * 5,
        compiler_params=pltpu.CompilerParams(
            dimension_semantics=("parallel", "arbitrary")),
        interpret=interpret,
    )(inst4, cls4, reg4, anc4, annotations, annT)
    il_s, cl_s, rl_s, npos, nposi = [o[:, 0, 0] for o in outs]
    il = (il_s / jnp.maximum(nposi, 1.0)).mean(keepdims=True)
    cl = (cl_s / jnp.maximum(npos, 1.0)).mean(keepdims=True)
    rl = (rl_s / jnp.maximum(npos * 4.0, 1.0)).mean(keepdims=True)
    return (il, cl, rl)


def kernel(instances, classifications, regressions, anchors, annotations):
    return _run(instances, classifications, regressions, anchors, annotations)
